# baseline (speedup 1.0000x reference)
import functools

import jax
import jax.numpy as jnp
from jax import lax
from jax.experimental import pallas as pl
from jax.experimental.pallas import tpu as pltpu
from jax.experimental.pallas import tpu_sc as plsc

N, E, D = 10000, 320000, 128
L = 16
NC, NS = 2, 16
NW = NC * NS
EPT = E // NW
C = 25
K = 4
S = EPT // (C * K)
PAIRS = S // 2
RPT = N // NS
DV = D // L
CW = 16


def _proj_body(x_ref, wr_ref, ws_ref, bm_ref, a_ref, b_ref):
    x = x_ref[...]
    dn = (((1,), (1,)), ((), ()))
    a_ref[...] = lax.dot_general(x, wr_ref[...], dn,
                                 preferred_element_type=jnp.float32) + bm_ref[...]
    b_ref[...] = lax.dot_general(x, ws_ref[...], dn,
                                 preferred_element_type=jnp.float32)


def _proj(x, wr, ws, bm):
    blk = 1000
    grid = N // blk
    return pl.pallas_call(
        _proj_body,
        grid=(grid,),
        in_specs=[
            pl.BlockSpec((blk, D), lambda i: (i, 0)),
            pl.BlockSpec((D, D), lambda i: (0, 0)),
            pl.BlockSpec((D, D), lambda i: (0, 0)),
            pl.BlockSpec((1, D), lambda i: (0, 0)),
        ],
        out_specs=[
            pl.BlockSpec((blk, D), lambda i: (i, 0)),
            pl.BlockSpec((blk, D), lambda i: (i, 0)),
        ],
        out_shape=[
            jax.ShapeDtypeStruct((N, D), jnp.float32),
            jax.ShapeDtypeStruct((N, D), jnp.float32),
        ],
    )(x, wr, ws, bm)


def _edge_sc(a, b, idx6, gb):
    mesh = plsc.VectorSubcoreMesh(core_axis_name="c", subcore_axis_name="s")

    @functools.partial(
        pl.kernel,
        out_type=jax.ShapeDtypeStruct((NC, NS, RPT, D), jnp.float32),
        mesh=mesh,
        compiler_params=pltpu.CompilerParams(needs_layout_passes=False),
        scratch_types=(
            [pltpu.VMEM((2, K, 2, C), jnp.int32)]
            + [pltpu.VMEM((C, D), jnp.float32) for _ in range(K)]
            + [pltpu.VMEM((C, D), jnp.float32) for _ in range(K)]
            + [pltpu.VMEM((2, D), jnp.float32),
               pltpu.VMEM_SHARED((N, D), jnp.float32)]
            + [pltpu.SemaphoreType.DMA] * (3 * K + 2)
        ),
    )
    def edge_kernel(a_hbm, b_hbm, idx_hbm, gb_hbm, zout_hbm, *refs):
        idxb = refs[0]
        abufs = refs[1:1 + K]
        bbufs = refs[1 + K:1 + 2 * K]
        gb_v = refs[1 + 2 * K]
        inbox_sh = refs[2 + 2 * K]
        sems = refs[3 + 2 * K:]
        ga = sems[0:K]
        gbm = sems[K:2 * K]
        ss = sems[2 * K:3 * K]
        si = sems[3 * K:3 * K + 2]

        cid = lax.axis_index("c")
        sid = lax.axis_index("s")
        wid = sid * NC + cid

        pltpu.sync_copy(gb_hbm, gb_v)
        gs = [gb_v[0, pl.ds(j * L, L)] for j in range(DV)]
        bs = [gb_v[1, pl.ds(j * L, L)] for j in range(DV)]

        def fire_gather(p, k, bk):
            pltpu.async_copy(a_hbm.at[idxb.at[p, k, 0]], abufs[bk], ga[bk])
            pltpu.async_copy(b_hbm.at[idxb.at[p, k, 1]], bbufs[bk], gbm[bk])

        def wait_gather(p, k, bk):
            pltpu.make_async_copy(a_hbm.at[idxb.at[p, k, 0]], abufs[bk],
                                  ga[bk]).wait()
            pltpu.make_async_copy(b_hbm.at[idxb.at[p, k, 1]], bbufs[bk],
                                  gbm[bk]).wait()

        def fire_scatter(p, k, bk):
            pltpu.async_copy(abufs[bk], inbox_sh.at[idxb.at[p, k, 0]],
                             ss[bk], add=True)

        def wait_scatter(p, k, bk):
            pltpu.make_async_copy(abufs[bk], inbox_sh.at[idxb.at[p, k, 0]],
                                  ss[bk]).wait()

        pltpu.sync_copy(idx_hbm.at[wid, 0], idxb.at[0])
        fire_gather(0, 0, 0)
        fire_gather(0, 1, 1)

        zb = bbufs[K - 1]

        def zrow(i, carry):
            for j in range(DV):
                zb[i, pl.ds(j * L, L)] = jnp.zeros((L,), jnp.float32)
            return carry
        lax.fori_loop(0, C, zrow, 0)
        r0 = sid * RPT

        def zslab(kk, carry):
            pltpu.sync_copy(zb, inbox_sh.at[pl.ds(r0 + kk * C, C)])
            return carry
        lax.fori_loop(0, RPT // C, zslab, 0)
        _rem = RPT - (RPT // C) * C
        if _rem:
            pltpu.sync_copy(zb.at[pl.ds(0, _rem)],
                            inbox_sh.at[pl.ds(r0 + (RPT // C) * C, _rem)])
        plsc.subcore_barrier()

        inv_d = jnp.float32(1.0 / D)

        def make_edge(av_ref, bv_ref):
            def edge(e):
                ts = []
                for j in range(DV):
                    av = av_ref[e, pl.ds(j * L, L)]
                    bv = bv_ref[e, pl.ds(j * L, L)]
                    ts.append(av + bv)
                s_v = ts[0]
                for j in range(1, DV):
                    s_v = s_v + ts[j]
                q_v = ts[0] * ts[0]
                for j in range(1, DV):
                    q_v = q_v + ts[j] * ts[j]
                mu = jnp.full((L,), jnp.sum(s_v)) * inv_d
                ex2 = jnp.full((L,), jnp.sum(q_v)) * inv_d
                var = jnp.maximum(ex2 - mu * mu, 0.0) + jnp.float32(1e-5)
                iv = plsc.bitcast(var, jnp.int32)
                iv = jnp.int32(0x5F3759DF) - lax.shift_right_arithmetic(iv, 1)
                y = plsc.bitcast(iv, jnp.float32)
                half_v = jnp.float32(0.5) * var
                y = y * (jnp.float32(1.5) - half_v * y * y)
                y = y * (jnp.float32(1.5) - half_v * y * y)
                for j in range(DV):
                    z = (ts[j] - mu) * y
                    av_ref[e, pl.ds(j * L, L)] = z * gs[j] + bs[j]
            return edge

        edge_fns = [make_edge(abufs[k], bbufs[k]) for k in range(K)]

        def pair_body(t, carry):
            for p in range(2):
                for k in range(K):
                    wait_gather(p, k, k)
                    plsc.parallel_loop(0, C, unroll=3)(edge_fns[k])
                    fire_scatter(p, k, k)
                    sp = p if k >= K - 2 else (p + 1) % 2
                    if p == 0 and k < K - 2:
                        @pl.when(t > 0)
                        def _():
                            wait_scatter(sp, (k + 2) % K, (k + 2) % K)
                    else:
                        wait_scatter(sp, (k + 2) % K, (k + 2) % K)
                    if k == K - 3:
                        if p == 0:
                            pltpu.async_copy(idx_hbm.at[wid, 2 * t + 1],
                                             idxb.at[1], si[1])
                        else:
                            @pl.when(t < PAIRS - 1)
                            def _():
                                pltpu.async_copy(idx_hbm.at[wid, 2 * t + 2],
                                                 idxb.at[0], si[0])
                    if k == K - 2:
                        if p == 0:
                            pltpu.make_async_copy(idx_hbm.at[wid, 2 * t + 1],
                                                  idxb.at[1], si[1]).wait()
                        else:
                            @pl.when(t < PAIRS - 1)
                            def _():
                                pltpu.make_async_copy(
                                    idx_hbm.at[wid, 2 * t + 2],
                                    idxb.at[0], si[0]).wait()
                    if k < K - 2:
                        fire_gather(p, k + 2, (k + 2) % K)
                    elif p == 0:
                        fire_gather(1, k - (K - 2), (k + 2) % K)
                    else:
                        @pl.when(t < PAIRS - 1)
                        def _():
                            fire_gather(0, k - (K - 2), (k + 2) % K)
            return carry

        lax.fori_loop(0, PAIRS, pair_body, 0)

        for k in range(2, K):
            wait_scatter(1, k, k)
        plsc.subcore_barrier()

        pltpu.sync_copy(inbox_sh.at[pl.ds(r0, RPT)], zout_hbm.at[cid, sid])

    return edge_kernel(a, b, idx6, gb)


def _node_body(x_ref, z_ref, w1_ref, w2_ref, bn_ref, g_ref, b2_ref, o_ref):
    x = x_ref[...]
    ib = z_ref[0] + z_ref[1]
    dn = (((1,), (1,)), ((), ()))
    h = (lax.dot_general(x, w1_ref[...], dn, preferred_element_type=jnp.float32)
         + lax.dot_general(ib, w2_ref[...], dn, preferred_element_type=jnp.float32)
         + bn_ref[...])
    mu = jnp.mean(h, axis=-1, keepdims=True)
    var = jnp.mean((h - mu) ** 2, axis=-1, keepdims=True)
    o_ref[...] = (h - mu) * lax.rsqrt(var + 1e-5) * g_ref[...] + b2_ref[...]


def _node(x, zparts, w1, w2, bn, g2, b2):
    blk = 1000
    grid = N // blk
    return pl.pallas_call(
        _node_body,
        grid=(grid,),
        in_specs=[
            pl.BlockSpec((blk, D), lambda i: (i, 0)),
            pl.BlockSpec((NC, blk, D), lambda i: (0, i, 0)),
            pl.BlockSpec((D, D), lambda i: (0, 0)),
            pl.BlockSpec((D, D), lambda i: (0, 0)),
            pl.BlockSpec((1, D), lambda i: (0, 0)),
            pl.BlockSpec((1, D), lambda i: (0, 0)),
            pl.BlockSpec((1, D), lambda i: (0, 0)),
        ],
        out_specs=pl.BlockSpec((blk, D), lambda i: (i, 0)),
        out_shape=jax.ShapeDtypeStruct((N, D), jnp.float32),
    )(x, zparts, w1, w2, bn, g2, b2)


def kernel(nodes, senders, receivers, W_msg, b_msg, ln1_g, ln1_b,
           W_node, b_node, ln2_g, ln2_b):
    x = nodes[0]
    wr = W_msg[:, :D]
    ws = W_msg[:, D:]
    wn1 = W_node[:, :D]
    wn2 = W_node[:, D:]
    a, b = _proj(x, wr, ws, b_msg.reshape(1, D))
    recv5 = receivers.reshape(NW, S, K, 1, C)
    send5 = senders.reshape(NW, S, K, 1, C)
    idx6 = jnp.concatenate([recv5, send5], axis=3)
    gb = jnp.stack([ln1_g, ln1_b], axis=0)
    zparts = _edge_sc(a, b, idx6, gb).reshape(NC, N, D)
    out = _node(x, zparts, wn1, wn2, b_node.reshape(1, D),
                ln2_g.reshape(1, D), ln2_b.reshape(1, D))
    return out[None]


# speedup vs baseline: 1.1771x; 1.1771x over previous
import functools

import jax
import jax.numpy as jnp
from jax import lax
from jax.experimental import pallas as pl
from jax.experimental.pallas import tpu as pltpu
from jax.experimental.pallas import tpu_sc as plsc

N, E, D = 10000, 320000, 128
L = 16
NC, NS = 2, 16
NW = NC * NS
EPT = E // NW
C = 25
K = 4
S = EPT // (C * K)
PAIRS = S // 2
RPT = N // NS
DV = D // L
CW = 16


def _proj_body(x_ref, wr_ref, ws_ref, bm_ref, a_ref, b_ref):
    x = x_ref[...]
    dn = (((1,), (1,)), ((), ()))
    a_ref[...] = lax.dot_general(x, wr_ref[...], dn,
                                 preferred_element_type=jnp.float32) + bm_ref[...]
    b_ref[...] = lax.dot_general(x, ws_ref[...], dn,
                                 preferred_element_type=jnp.float32)


def _proj(x, wr, ws, bm):
    blk = 1000
    grid = N // blk
    return pl.pallas_call(
        _proj_body,
        grid=(grid,),
        in_specs=[
            pl.BlockSpec((blk, D), lambda i: (i, 0)),
            pl.BlockSpec((D, D), lambda i: (0, 0)),
            pl.BlockSpec((D, D), lambda i: (0, 0)),
            pl.BlockSpec((1, D), lambda i: (0, 0)),
        ],
        out_specs=[
            pl.BlockSpec((blk, D), lambda i: (i, 0)),
            pl.BlockSpec((blk, D), lambda i: (i, 0)),
        ],
        out_shape=[
            jax.ShapeDtypeStruct((N, D), jnp.float32),
            jax.ShapeDtypeStruct((N, D), jnp.float32),
        ],
    )(x, wr, ws, bm)


def _edge_sc(a, b, idx6, gb):
    mesh = plsc.VectorSubcoreMesh(core_axis_name="c", subcore_axis_name="s")

    @functools.partial(
        pl.kernel,
        out_type=jax.ShapeDtypeStruct((NC, NS, RPT, D), jnp.float32),
        mesh=mesh,
        compiler_params=pltpu.CompilerParams(needs_layout_passes=False),
        scratch_types=(
            [pltpu.VMEM((2, K, 2, C), jnp.int32)]
            + [pltpu.VMEM((C, D), jnp.float32) for _ in range(K)]
            + [pltpu.VMEM((C, D), jnp.float32) for _ in range(K)]
            + [pltpu.VMEM((2, D), jnp.float32),
               pltpu.VMEM_SHARED((N, D), jnp.float32)]
            + [pltpu.SemaphoreType.DMA] * (3 * K + 2)
        ),
    )
    def edge_kernel(a_hbm, b_hbm, idx_hbm, gb_hbm, zout_hbm, *refs):
        idxb = refs[0]
        abufs = refs[1:1 + K]
        bbufs = refs[1 + K:1 + 2 * K]
        gb_v = refs[1 + 2 * K]
        inbox_sh = refs[2 + 2 * K]
        sems = refs[3 + 2 * K:]
        ga = sems[0:K]
        gbm = sems[K:2 * K]
        ss = sems[2 * K:3 * K]
        si = sems[3 * K:3 * K + 2]

        cid = lax.axis_index("c")
        sid = lax.axis_index("s")
        wid = sid * NC + cid

        pltpu.sync_copy(gb_hbm, gb_v)
        gs = [gb_v[0, pl.ds(j * L, L)] for j in range(DV)]
        bs = [gb_v[1, pl.ds(j * L, L)] for j in range(DV)]

        def fire_gather(p, k, bk):
            pltpu.async_copy(a_hbm.at[idxb.at[p, k, 0]], abufs[bk], ga[bk])
            pltpu.async_copy(b_hbm.at[idxb.at[p, k, 1]], bbufs[bk], gbm[bk])

        def wait_gather(p, k, bk):
            pltpu.make_async_copy(a_hbm.at[idxb.at[p, k, 0]], abufs[bk],
                                  ga[bk]).wait()
            pltpu.make_async_copy(b_hbm.at[idxb.at[p, k, 1]], bbufs[bk],
                                  gbm[bk]).wait()

        def fire_scatter(p, k, bk):
            pltpu.async_copy(abufs[bk], inbox_sh.at[idxb.at[p, k, 0]],
                             ss[bk], add=True)

        def wait_scatter(p, k, bk):
            pltpu.make_async_copy(abufs[bk], inbox_sh.at[idxb.at[p, k, 0]],
                                  ss[bk]).wait()

        pltpu.sync_copy(idx_hbm.at[wid, 0], idxb.at[0])
        fire_gather(0, 0, 0)
        fire_gather(0, 1, 1)

        zb = bbufs[K - 1]

        def zrow(i, carry):
            for j in range(DV):
                zb[i, pl.ds(j * L, L)] = jnp.zeros((L,), jnp.float32)
            return carry
        lax.fori_loop(0, C, zrow, 0)
        r0 = sid * RPT

        def zslab(kk, carry):
            pltpu.sync_copy(zb, inbox_sh.at[pl.ds(r0 + kk * C, C)])
            return carry
        lax.fori_loop(0, RPT // C, zslab, 0)
        _rem = RPT - (RPT // C) * C
        if _rem:
            pltpu.sync_copy(zb.at[pl.ds(0, _rem)],
                            inbox_sh.at[pl.ds(r0 + (RPT // C) * C, _rem)])
        plsc.subcore_barrier()

        inv_d = jnp.float32(1.0 / D)

        def make_edge(av_ref, bv_ref):
            def edge(e):
                ts = []
                for j in range(DV):
                    av = av_ref[e, pl.ds(j * L, L)]
                    bv = bv_ref[e, pl.ds(j * L, L)]
                    ts.append(av + bv)
                s_v = ts[0]
                for j in range(1, DV):
                    s_v = s_v + ts[j]
                q_v = ts[0] * ts[0]
                for j in range(1, DV):
                    q_v = q_v + ts[j] * ts[j]
                mu = jnp.full((L,), jnp.sum(s_v)) * inv_d
                ex2 = jnp.full((L,), jnp.sum(q_v)) * inv_d
                var = jnp.maximum(ex2 - mu * mu, 0.0) + jnp.float32(1e-5)
                iv = plsc.bitcast(var, jnp.int32)
                iv = jnp.int32(0x5F3759DF) - lax.shift_right_arithmetic(iv, 1)
                y = plsc.bitcast(iv, jnp.float32)
                half_v = jnp.float32(0.5) * var
                y = y * (jnp.float32(1.5) - half_v * y * y)
                for j in range(DV):
                    z = (ts[j] - mu) * y
                    av_ref[e, pl.ds(j * L, L)] = z * gs[j] + bs[j]
            return edge

        edge_fns = [make_edge(abufs[k], bbufs[k]) for k in range(K)]

        def pair_body(t, carry):
            for p in range(2):
                for k in range(K):
                    wait_gather(p, k, k)
                    plsc.parallel_loop(0, C, unroll=2)(edge_fns[k])
                    fire_scatter(p, k, k)
                    sp = p if k >= K - 2 else (p + 1) % 2
                    if p == 0 and k < K - 2:
                        @pl.when(t > 0)
                        def _():
                            wait_scatter(sp, (k + 2) % K, (k + 2) % K)
                    else:
                        wait_scatter(sp, (k + 2) % K, (k + 2) % K)
                    if k == K - 3:
                        if p == 0:
                            pltpu.async_copy(idx_hbm.at[wid, 2 * t + 1],
                                             idxb.at[1], si[1])
                        else:
                            @pl.when(t < PAIRS - 1)
                            def _():
                                pltpu.async_copy(idx_hbm.at[wid, 2 * t + 2],
                                                 idxb.at[0], si[0])
                    if k == K - 2:
                        if p == 0:
                            pltpu.make_async_copy(idx_hbm.at[wid, 2 * t + 1],
                                                  idxb.at[1], si[1]).wait()
                        else:
                            @pl.when(t < PAIRS - 1)
                            def _():
                                pltpu.make_async_copy(
                                    idx_hbm.at[wid, 2 * t + 2],
                                    idxb.at[0], si[0]).wait()
                    if k < K - 2:
                        fire_gather(p, k + 2, (k + 2) % K)
                    elif p == 0:
                        fire_gather(1, k - (K - 2), (k + 2) % K)
                    else:
                        @pl.when(t < PAIRS - 1)
                        def _():
                            fire_gather(0, k - (K - 2), (k + 2) % K)
            return carry

        lax.fori_loop(0, PAIRS, pair_body, 0)

        for k in range(2, K):
            wait_scatter(1, k, k)
        plsc.subcore_barrier()

        pltpu.sync_copy(inbox_sh.at[pl.ds(r0, RPT)], zout_hbm.at[cid, sid])

    return edge_kernel(a, b, idx6, gb)


def _node_body(x_ref, z_ref, w1_ref, w2_ref, bn_ref, g_ref, b2_ref, o_ref):
    x = x_ref[...]
    ib = z_ref[0] + z_ref[1]
    dn = (((1,), (1,)), ((), ()))
    h = (lax.dot_general(x, w1_ref[...], dn, preferred_element_type=jnp.float32)
         + lax.dot_general(ib, w2_ref[...], dn, preferred_element_type=jnp.float32)
         + bn_ref[...])
    mu = jnp.mean(h, axis=-1, keepdims=True)
    var = jnp.mean((h - mu) ** 2, axis=-1, keepdims=True)
    o_ref[...] = (h - mu) * lax.rsqrt(var + 1e-5) * g_ref[...] + b2_ref[...]


def _node(x, zparts, w1, w2, bn, g2, b2):
    blk = 1000
    grid = N // blk
    return pl.pallas_call(
        _node_body,
        grid=(grid,),
        in_specs=[
            pl.BlockSpec((blk, D), lambda i: (i, 0)),
            pl.BlockSpec((NC, blk, D), lambda i: (0, i, 0)),
            pl.BlockSpec((D, D), lambda i: (0, 0)),
            pl.BlockSpec((D, D), lambda i: (0, 0)),
            pl.BlockSpec((1, D), lambda i: (0, 0)),
            pl.BlockSpec((1, D), lambda i: (0, 0)),
            pl.BlockSpec((1, D), lambda i: (0, 0)),
        ],
        out_specs=pl.BlockSpec((blk, D), lambda i: (i, 0)),
        out_shape=jax.ShapeDtypeStruct((N, D), jnp.float32),
    )(x, zparts, w1, w2, bn, g2, b2)


def kernel(nodes, senders, receivers, W_msg, b_msg, ln1_g, ln1_b,
           W_node, b_node, ln2_g, ln2_b):
    x = nodes[0]
    wr = W_msg[:, :D]
    ws = W_msg[:, D:]
    wn1 = W_node[:, :D]
    wn2 = W_node[:, D:]
    a, b = _proj(x, wr, ws, b_msg.reshape(1, D))
    recv5 = receivers.reshape(NW, S, K, 1, C)
    send5 = senders.reshape(NW, S, K, 1, C)
    idx6 = jnp.concatenate([recv5, send5], axis=3)
    gb = jnp.stack([ln1_g, ln1_b], axis=0)
    zparts = _edge_sc(a, b, idx6, gb).reshape(NC, N, D)
    out = _node(x, zparts, wn1, wn2, b_node.reshape(1, D),
                ln2_g.reshape(1, D), ln2_b.reshape(1, D))
    return out[None]
